# Initial kernel scaffold; baseline (speedup 1.0000x reference)
import functools

import jax
import jax.numpy as jnp
from jax import lax
from jax.experimental import pallas as pl
from jax.experimental.pallas import tpu as pltpu
from jax.experimental.pallas import tpu_sc as plsc

N = 10000
E = 160000
F_IN = 256
H = 256
C = 40

FH = 128
CP = 48
B_EDGE = 128
E_PAD = 163840
N_ROWS_IDX = E_PAD // B_EDGE
N_SUBCORES = 16
N_CORES = 2
N_WORKERS = 32
CHUNKS_L1 = N_ROWS_IDX // N_SUBCORES
CHUNKS_L2 = N_ROWS_IDX // N_WORKERS
N_PAD = 10240
ROWS_PER_TILE = N_PAD // N_SUBCORES
OUT_ROWS_PER_TILE = N // N_SUBCORES

_MESH = plsc.VectorSubcoreMesh(core_axis_name="c", subcore_axis_name="s")


def _l1_body(xa, xb, src_hbm, dst_hbm, z128, z16, ones_hbm,
             agg_a, agg_b, cnt_out,
             src_v, dst_v, rows_v, ones_v, acc, cacc, sem):
    cid = lax.axis_index("c")
    sid = lax.axis_index("s")
    r0 = sid * CHUNKS_L1
    pltpu.sync_copy(src_hbm.at[pl.ds(r0, CHUNKS_L1)], src_v)
    pltpu.sync_copy(dst_hbm.at[pl.ds(r0, CHUNKS_L1)], dst_v)
    pltpu.sync_copy(ones_hbm, ones_v)
    pltpu.sync_copy(z128, acc.at[pl.ds(sid * ROWS_PER_TILE, ROWS_PER_TILE)])

    @pl.when(cid == 0)
    def _():
        pltpu.sync_copy(z16, cacc.at[pl.ds(sid * ROWS_PER_TILE, ROWS_PER_TILE)])

    plsc.subcore_barrier()

    def _run(table, with_counts):
        def body(j, carry):
            pltpu.async_copy(table.at[src_v.at[j]], rows_v, sem).wait()
            pltpu.sync_copy(rows_v, acc.at[dst_v.at[j]], add=True)
            if with_counts:
                pltpu.sync_copy(ones_v, cacc.at[dst_v.at[j]], add=True)
            return carry
        lax.fori_loop(0, CHUNKS_L1, body, 0)

    @pl.when(cid == 0)
    def _():
        _run(xa, True)

    @pl.when(cid == 1)
    def _():
        _run(xb, False)

    plsc.subcore_barrier()

    o0 = sid * OUT_ROWS_PER_TILE

    @pl.when(cid == 0)
    def _():
        pltpu.sync_copy(acc.at[pl.ds(o0, OUT_ROWS_PER_TILE)],
                        agg_a.at[pl.ds(o0, OUT_ROWS_PER_TILE)])
        pltpu.sync_copy(cacc.at[pl.ds(o0, OUT_ROWS_PER_TILE)],
                        cnt_out.at[pl.ds(o0, OUT_ROWS_PER_TILE)])

    @pl.when(cid == 1)
    def _():
        pltpu.sync_copy(acc.at[pl.ds(o0, OUT_ROWS_PER_TILE)],
                        agg_b.at[pl.ds(o0, OUT_ROWS_PER_TILE)])


_l1_aggregate = pl.kernel(
    _l1_body,
    out_type=[
        jax.ShapeDtypeStruct((N, FH), jnp.float32),
        jax.ShapeDtypeStruct((N, FH), jnp.float32),
        jax.ShapeDtypeStruct((N, 16), jnp.float32),
    ],
    mesh=_MESH,
    scratch_types=[
        pltpu.VMEM((CHUNKS_L1, B_EDGE), jnp.int32),
        pltpu.VMEM((CHUNKS_L1, B_EDGE), jnp.int32),
        pltpu.VMEM((B_EDGE, FH), jnp.float32),
        pltpu.VMEM((B_EDGE, 16), jnp.float32),
        pltpu.VMEM_SHARED((N_PAD, FH), jnp.float32),
        pltpu.VMEM_SHARED((N_PAD, 16), jnp.float32),
        pltpu.SemaphoreType.DMA,
    ],
)


def _l2_body(p_hbm, src_hbm, dst_hbm, z48,
             agg2_a, agg2_b,
             src_v, dst_v, rows_v, acc, sem):
    cid = lax.axis_index("c")
    sid = lax.axis_index("s")
    wid = sid * N_CORES + cid
    r0 = wid * CHUNKS_L2
    pltpu.sync_copy(src_hbm.at[pl.ds(r0, CHUNKS_L2)], src_v)
    pltpu.sync_copy(dst_hbm.at[pl.ds(r0, CHUNKS_L2)], dst_v)
    pltpu.sync_copy(z48, acc.at[pl.ds(sid * ROWS_PER_TILE, ROWS_PER_TILE)])
    plsc.subcore_barrier()

    def body(j, carry):
        pltpu.async_copy(p_hbm.at[src_v.at[j]], rows_v, sem).wait()
        pltpu.sync_copy(rows_v, acc.at[dst_v.at[j]], add=True)
        return carry
    lax.fori_loop(0, CHUNKS_L2, body, 0)

    plsc.subcore_barrier()

    o0 = sid * OUT_ROWS_PER_TILE

    @pl.when(cid == 0)
    def _():
        pltpu.sync_copy(acc.at[pl.ds(o0, OUT_ROWS_PER_TILE)],
                        agg2_a.at[pl.ds(o0, OUT_ROWS_PER_TILE)])

    @pl.when(cid == 1)
    def _():
        pltpu.sync_copy(acc.at[pl.ds(o0, OUT_ROWS_PER_TILE)],
                        agg2_b.at[pl.ds(o0, OUT_ROWS_PER_TILE)])


_l2_aggregate = pl.kernel(
    _l2_body,
    out_type=[
        jax.ShapeDtypeStruct((N, CP), jnp.float32),
        jax.ShapeDtypeStruct((N, CP), jnp.float32),
    ],
    mesh=_MESH,
    scratch_types=[
        pltpu.VMEM((CHUNKS_L2, B_EDGE), jnp.int32),
        pltpu.VMEM((CHUNKS_L2, B_EDGE), jnp.int32),
        pltpu.VMEM((B_EDGE, CP), jnp.float32),
        pltpu.VMEM_SHARED((N_PAD, CP), jnp.float32),
        pltpu.SemaphoreType.DMA,
    ],
)

ROW_BLK = 2000


def _mlp_body(x_ref, agg_a_ref, agg_b_ref, cnt_ref,
              w1l_ref, b1_ref, w1r_ref, w2l_ref, b2_ref, w2r_ref,
              p_ref, q_ref):
    inv = 1.0 / jnp.maximum(cnt_ref[:, 0:1], 1.0)
    ha = agg_a_ref[...] * inv
    hb = agg_b_ref[...] * inv
    w1l = w1l_ref[...]
    s = jnp.dot(ha, w1l[0:FH, :], preferred_element_type=jnp.float32)
    s = s + jnp.dot(hb, w1l[FH:F_IN, :], preferred_element_type=jnp.float32)
    s = s + jnp.dot(x_ref[...], w1r_ref[...], preferred_element_type=jnp.float32)
    h = jnp.maximum(s + b1_ref[...], 0.0)
    p_ref[...] = jnp.dot(h, w2l_ref[...], preferred_element_type=jnp.float32)
    q_ref[...] = jnp.dot(h, w2r_ref[...], preferred_element_type=jnp.float32) + b2_ref[...]


def _out_body(a2a_ref, a2b_ref, cnt_ref, q_ref, o_ref):
    inv = 1.0 / jnp.maximum(cnt_ref[:, 0:1], 1.0)
    s = (a2a_ref[...] + a2b_ref[...]) * inv
    t = s[:, 0:C] + q_ref[...]
    m = jnp.max(t, axis=1, keepdims=True)
    lse = jnp.log(jnp.sum(jnp.exp(t - m), axis=1, keepdims=True)) + m
    o_ref[...] = t - lse


def kernel(x, edge_index, W1_l, b1_l, W1_r, W2_l, b2_l, W2_r):
    src = edge_index[0]
    dst = edge_index[1]
    pad = E_PAD - E
    src2 = jnp.concatenate([src, jnp.zeros((pad,), jnp.int32)]).reshape(N_ROWS_IDX, B_EDGE)
    dst2 = jnp.concatenate([dst, jnp.full((pad,), N, jnp.int32)]).reshape(N_ROWS_IDX, B_EDGE)
    xa = x[:, :FH]
    xb = x[:, FH:]
    z128 = jnp.zeros((ROWS_PER_TILE, FH), jnp.float32)
    z16 = jnp.zeros((ROWS_PER_TILE, 16), jnp.float32)
    z48 = jnp.zeros((ROWS_PER_TILE, CP), jnp.float32)
    ones16 = jnp.ones((B_EDGE, 16), jnp.float32)

    agg_a, agg_b, cnt16 = _l1_aggregate(xa, xb, src2, dst2, z128, z16, ones16)

    w2l_pad = jnp.concatenate([W2_l, jnp.zeros((H, CP - C), jnp.float32)], axis=1)
    b1r = b1_l.reshape(1, H)
    b2r = b2_l.reshape(1, C)

    n_blk = N // ROW_BLK
    p, q = pl.pallas_call(
        _mlp_body,
        grid=(n_blk,),
        in_specs=[
            pl.BlockSpec((ROW_BLK, F_IN), lambda i: (i, 0)),
            pl.BlockSpec((ROW_BLK, FH), lambda i: (i, 0)),
            pl.BlockSpec((ROW_BLK, FH), lambda i: (i, 0)),
            pl.BlockSpec((ROW_BLK, 16), lambda i: (i, 0)),
            pl.BlockSpec((F_IN, H), lambda i: (0, 0)),
            pl.BlockSpec((1, H), lambda i: (0, 0)),
            pl.BlockSpec((F_IN, H), lambda i: (0, 0)),
            pl.BlockSpec((H, CP), lambda i: (0, 0)),
            pl.BlockSpec((1, C), lambda i: (0, 0)),
            pl.BlockSpec((H, C), lambda i: (0, 0)),
        ],
        out_specs=[
            pl.BlockSpec((ROW_BLK, CP), lambda i: (i, 0)),
            pl.BlockSpec((ROW_BLK, C), lambda i: (i, 0)),
        ],
        out_shape=[
            jax.ShapeDtypeStruct((N, CP), jnp.float32),
            jax.ShapeDtypeStruct((N, C), jnp.float32),
        ],
    )(x, agg_a, agg_b, cnt16, W1_l, b1r, W1_r, w2l_pad, b2r, W2_r)

    agg2_a, agg2_b = _l2_aggregate(p, src2, dst2, z48)

    out = pl.pallas_call(
        _out_body,
        grid=(n_blk,),
        in_specs=[
            pl.BlockSpec((ROW_BLK, CP), lambda i: (i, 0)),
            pl.BlockSpec((ROW_BLK, CP), lambda i: (i, 0)),
            pl.BlockSpec((ROW_BLK, 16), lambda i: (i, 0)),
            pl.BlockSpec((ROW_BLK, C), lambda i: (i, 0)),
        ],
        out_specs=pl.BlockSpec((ROW_BLK, C), lambda i: (i, 0)),
        out_shape=jax.ShapeDtypeStruct((N, C), jnp.float32),
    )(agg2_a, agg2_b, cnt16, q)
    return out


# speedup vs baseline: 3.4608x; 3.4608x over previous
import jax
import jax.numpy as jnp
from jax import lax
from jax.experimental import pallas as pl
from jax.experimental.pallas import tpu as pltpu
from jax.experimental.pallas import tpu_sc as plsc

N = 10000
E = 160000
F_IN = 256
H = 256
C = 40

FH = 128
CP = 128
B_EDGE = 128
E_PAD = 163840
N_ROWS_IDX = E_PAD // B_EDGE
N_SUBCORES = 16
N_CORES = 2
CHUNKS_L1 = N_ROWS_IDX // N_SUBCORES
HALF_L1 = CHUNKS_L1 // 2
CHUNKS_L2 = N_ROWS_IDX // (N_SUBCORES * N_CORES)
N_PAD = 10112
ROWS_PER_TILE = N_PAD // N_SUBCORES

_MESH = plsc.VectorSubcoreMesh(core_axis_name="c", subcore_axis_name="s")


def _cnt_body(dst_hbm, z128, ones_hbm,
              cnt_a, cnt_b,
              dst_v, ones_v, cacc):
    cid = lax.axis_index("c")
    sid = lax.axis_index("s")
    wid = sid * N_CORES + cid
    r0 = wid * CHUNKS_L2
    sl = pl.ds(sid * ROWS_PER_TILE, ROWS_PER_TILE)
    pltpu.sync_copy(dst_hbm.at[pl.ds(r0, CHUNKS_L2)], dst_v)
    pltpu.sync_copy(z128, cacc.at[sl])
    pltpu.sync_copy(ones_hbm, ones_v)
    plsc.subcore_barrier()

    def body(j, carry):
        pltpu.sync_copy(ones_v, cacc.at[dst_v.at[j]], add=True)
        return carry
    lax.fori_loop(0, CHUNKS_L2, body, 0)

    plsc.subcore_barrier()

    @pl.when(cid == 0)
    def _():
        pltpu.sync_copy(cacc.at[sl], cnt_a.at[sl])

    @pl.when(cid == 1)
    def _():
        pltpu.sync_copy(cacc.at[sl], cnt_b.at[sl])


_cnt_aggregate = pl.kernel(
    _cnt_body,
    out_type=[
        jax.ShapeDtypeStruct((N_PAD, 128), jnp.float32),
        jax.ShapeDtypeStruct((N_PAD, 128), jnp.float32),
    ],
    mesh=_MESH,
    scratch_types=[
        pltpu.VMEM((CHUNKS_L2, B_EDGE), jnp.int32),
        pltpu.VMEM((B_EDGE, 128), jnp.float32),
        pltpu.VMEM_SHARED((N_PAD, 128), jnp.float32),
    ],
)


def _l1_body(xa, xb, src_hbm, dst_hbm, z128,
             agg_a, agg_b,
             src_v, dst_v, rows_v, acc, sem):
    cid = lax.axis_index("c")
    sid = lax.axis_index("s")
    r0 = sid * CHUNKS_L1
    sl = pl.ds(sid * ROWS_PER_TILE, ROWS_PER_TILE)
    pltpu.sync_copy(z128, acc.at[sl])

    plsc.subcore_barrier()

    def _run(table):
        def body(j, carry):
            pltpu.async_copy(table.at[src_v.at[j]], rows_v, sem).wait()
            pltpu.sync_copy(rows_v, acc.at[dst_v.at[j]], add=True)
            return carry
        for half in range(2):
            r = r0 + half * HALF_L1
            pltpu.sync_copy(src_hbm.at[pl.ds(r, HALF_L1)], src_v)
            pltpu.sync_copy(dst_hbm.at[pl.ds(r, HALF_L1)], dst_v)
            lax.fori_loop(0, HALF_L1, body, 0)

    @pl.when(cid == 0)
    def _():
        _run(xa)

    @pl.when(cid == 1)
    def _():
        _run(xb)

    plsc.subcore_barrier()

    @pl.when(cid == 0)
    def _():
        pltpu.sync_copy(acc.at[sl], agg_a.at[sl])

    @pl.when(cid == 1)
    def _():
        pltpu.sync_copy(acc.at[sl], agg_b.at[sl])


_l1_aggregate = pl.kernel(
    _l1_body,
    out_type=[
        jax.ShapeDtypeStruct((N_PAD, FH), jnp.float32),
        jax.ShapeDtypeStruct((N_PAD, FH), jnp.float32),
    ],
    mesh=_MESH,
    scratch_types=[
        pltpu.VMEM((HALF_L1, B_EDGE), jnp.int32),
        pltpu.VMEM((HALF_L1, B_EDGE), jnp.int32),
        pltpu.VMEM((B_EDGE, FH), jnp.float32),
        pltpu.VMEM_SHARED((N_PAD, FH), jnp.float32),
        pltpu.SemaphoreType.DMA,
    ],
)


def _l2_body(p_hbm, src_hbm, dst_hbm, z128,
             agg2_a, agg2_b,
             src_v, dst_v, rows_v, acc, sem):
    cid = lax.axis_index("c")
    sid = lax.axis_index("s")
    wid = sid * N_CORES + cid
    r0 = wid * CHUNKS_L2
    sl = pl.ds(sid * ROWS_PER_TILE, ROWS_PER_TILE)
    pltpu.sync_copy(src_hbm.at[pl.ds(r0, CHUNKS_L2)], src_v)
    pltpu.sync_copy(dst_hbm.at[pl.ds(r0, CHUNKS_L2)], dst_v)
    pltpu.sync_copy(z128, acc.at[sl])
    plsc.subcore_barrier()

    def body(j, carry):
        pltpu.async_copy(p_hbm.at[src_v.at[j]], rows_v, sem).wait()
        pltpu.sync_copy(rows_v, acc.at[dst_v.at[j]], add=True)
        return carry
    lax.fori_loop(0, CHUNKS_L2, body, 0)

    plsc.subcore_barrier()

    @pl.when(cid == 0)
    def _():
        pltpu.sync_copy(acc.at[sl], agg2_a.at[sl])

    @pl.when(cid == 1)
    def _():
        pltpu.sync_copy(acc.at[sl], agg2_b.at[sl])


_l2_aggregate = pl.kernel(
    _l2_body,
    out_type=[
        jax.ShapeDtypeStruct((N_PAD, CP), jnp.float32),
        jax.ShapeDtypeStruct((N_PAD, CP), jnp.float32),
    ],
    mesh=_MESH,
    scratch_types=[
        pltpu.VMEM((CHUNKS_L2, B_EDGE), jnp.int32),
        pltpu.VMEM((CHUNKS_L2, B_EDGE), jnp.int32),
        pltpu.VMEM((B_EDGE, CP), jnp.float32),
        pltpu.VMEM_SHARED((N_PAD, CP), jnp.float32),
        pltpu.SemaphoreType.DMA,
    ],
)

ROW_BLK = 632
OUT_BLK = 2000


def _mlp_body(x_ref, agg_a_ref, agg_b_ref, ca_ref, cb_ref,
              w1l_ref, b1_ref, w1r_ref, w2l_ref, b2_ref, w2r_ref,
              p_ref, q_ref):
    inv = 1.0 / jnp.maximum(ca_ref[...] + cb_ref[...], 1.0)
    ha = agg_a_ref[...] * inv
    hb = agg_b_ref[...] * inv
    w1l = w1l_ref[...]
    s = jnp.dot(ha, w1l[0:FH, :], preferred_element_type=jnp.float32)
    s = s + jnp.dot(hb, w1l[FH:F_IN, :], preferred_element_type=jnp.float32)
    s = s + jnp.dot(x_ref[...], w1r_ref[...], preferred_element_type=jnp.float32)
    h = jnp.maximum(s + b1_ref[...], 0.0)
    p_ref[...] = jnp.dot(h, w2l_ref[...], preferred_element_type=jnp.float32)
    q_ref[...] = jnp.dot(h, w2r_ref[...], preferred_element_type=jnp.float32) + b2_ref[...]


def _out_body(a2a_ref, a2b_ref, ca_ref, cb_ref, q_ref, o_ref):
    inv = 1.0 / jnp.maximum(ca_ref[...] + cb_ref[...], 1.0)
    s = (a2a_ref[...] + a2b_ref[...]) * inv
    t = s[:, 0:C] + q_ref[...]
    m = jnp.max(t, axis=1, keepdims=True)
    lse = jnp.log(jnp.sum(jnp.exp(t - m), axis=1, keepdims=True)) + m
    o_ref[...] = t - lse


def kernel(x, edge_index, W1_l, b1_l, W1_r, W2_l, b2_l, W2_r):
    src = edge_index[0]
    dst = edge_index[1]
    pad = E_PAD - E
    src2 = jnp.concatenate([src, jnp.zeros((pad,), jnp.int32)]).reshape(N_ROWS_IDX, B_EDGE)
    dst2 = jnp.concatenate([dst, jnp.full((pad,), N, jnp.int32)]).reshape(N_ROWS_IDX, B_EDGE)
    xa = x[:, :FH]
    xb = x[:, FH:]
    z128 = jnp.zeros((ROWS_PER_TILE, FH), jnp.float32)
    ones128 = jnp.ones((B_EDGE, 128), jnp.float32)

    cnt_a2, cnt_b2 = _cnt_aggregate(dst2, z128, ones128)
    ca = cnt_a2[:, 0:1]
    cb = cnt_b2[:, 0:1]
    agg_a, agg_b = _l1_aggregate(xa, xb, src2, dst2, z128)

    w2l_pad = jnp.concatenate([W2_l, jnp.zeros((H, CP - C), jnp.float32)], axis=1)
    b1r = b1_l.reshape(1, H)
    b2r = b2_l.reshape(1, C)

    p, q = pl.pallas_call(
        _mlp_body,
        grid=(N_PAD // ROW_BLK,),
        in_specs=[
            pl.BlockSpec((ROW_BLK, F_IN), lambda i: (i, 0)),
            pl.BlockSpec((ROW_BLK, FH), lambda i: (i, 0)),
            pl.BlockSpec((ROW_BLK, FH), lambda i: (i, 0)),
            pl.BlockSpec((ROW_BLK, 1), lambda i: (i, 0)),
            pl.BlockSpec((ROW_BLK, 1), lambda i: (i, 0)),
            pl.BlockSpec((F_IN, H), lambda i: (0, 0)),
            pl.BlockSpec((1, H), lambda i: (0, 0)),
            pl.BlockSpec((F_IN, H), lambda i: (0, 0)),
            pl.BlockSpec((H, CP), lambda i: (0, 0)),
            pl.BlockSpec((1, C), lambda i: (0, 0)),
            pl.BlockSpec((H, C), lambda i: (0, 0)),
        ],
        out_specs=[
            pl.BlockSpec((ROW_BLK, CP), lambda i: (i, 0)),
            pl.BlockSpec((ROW_BLK, C), lambda i: (i, 0)),
        ],
        out_shape=[
            jax.ShapeDtypeStruct((N_PAD, CP), jnp.float32),
            jax.ShapeDtypeStruct((N_PAD, C), jnp.float32),
        ],
    )(x, agg_a, agg_b, ca, cb, W1_l, b1r, W1_r, w2l_pad, b2r, W2_r)

    agg2_a, agg2_b = _l2_aggregate(p, src2, dst2, z128)

    out = pl.pallas_call(
        _out_body,
        grid=(N // OUT_BLK,),
        in_specs=[
            pl.BlockSpec((OUT_BLK, CP), lambda i: (i, 0)),
            pl.BlockSpec((OUT_BLK, CP), lambda i: (i, 0)),
            pl.BlockSpec((OUT_BLK, 1), lambda i: (i, 0)),
            pl.BlockSpec((OUT_BLK, 1), lambda i: (i, 0)),
            pl.BlockSpec((OUT_BLK, C), lambda i: (i, 0)),
        ],
        out_specs=pl.BlockSpec((OUT_BLK, C), lambda i: (i, 0)),
        out_shape=jax.ShapeDtypeStruct((N, C), jnp.float32),
    )(agg2_a, agg2_b, ca, cb, q)
    return out


# speedup vs baseline: 8.8558x; 2.5589x over previous
import jax
import jax.numpy as jnp
from jax import lax
from jax.experimental import pallas as pl
from jax.experimental.pallas import tpu as pltpu
from jax.experimental.pallas import tpu_sc as plsc

N = 10000
E = 160000
F_IN = 256
H = 256
C = 40

FH = 128
CP = 128
B_EDGE = 128
E_PAD = 163840
N_ROWS_IDX = E_PAD // B_EDGE
N_SUBCORES = 16
N_CORES = 2
CHUNKS_L1 = N_ROWS_IDX // N_SUBCORES
HALF_L1 = CHUNKS_L1 // 2
CHUNKS_L2 = N_ROWS_IDX // (N_SUBCORES * N_CORES)
STAGE = 40
N_PAD = 10112
ROWS_PER_TILE = N_PAD // N_SUBCORES

_MESH = plsc.VectorSubcoreMesh(core_axis_name="c", subcore_axis_name="s")


def _cnt_body(dst_hbm, z128, ones_hbm,
              cnt_a, cnt_b,
              dst_v, ones_v, cacc):
    cid = lax.axis_index("c")
    sid = lax.axis_index("s")
    wid = sid * N_CORES + cid
    r0 = wid * CHUNKS_L2
    sl = pl.ds(sid * ROWS_PER_TILE, ROWS_PER_TILE)
    pltpu.sync_copy(dst_hbm.at[pl.ds(r0, CHUNKS_L2)], dst_v)
    pltpu.sync_copy(z128, cacc.at[sl])
    pltpu.sync_copy(ones_hbm, ones_v)
    plsc.subcore_barrier()

    def body(j, carry):
        pltpu.sync_copy(ones_v, cacc.at[dst_v.at[j]], add=True)
        return carry
    lax.fori_loop(0, CHUNKS_L2, body, 0)

    plsc.subcore_barrier()

    @pl.when(cid == 0)
    def _():
        pltpu.sync_copy(cacc.at[sl], cnt_a.at[sl])

    @pl.when(cid == 1)
    def _():
        pltpu.sync_copy(cacc.at[sl], cnt_b.at[sl])


_cnt_aggregate = pl.kernel(
    _cnt_body,
    out_type=[
        jax.ShapeDtypeStruct((N_PAD, 128), jnp.float32),
        jax.ShapeDtypeStruct((N_PAD, 128), jnp.float32),
    ],
    mesh=_MESH,
    scratch_types=[
        pltpu.VMEM((CHUNKS_L2, B_EDGE), jnp.int32),
        pltpu.VMEM((B_EDGE, 128), jnp.float32),
        pltpu.VMEM_SHARED((N_PAD, 128), jnp.float32),
    ],
)


def _stream_stage(table, src_hbm, dst_hbm, r, acc, src_v, dst_v, rows_a, rows_b, s0, s1):
    r = pl.multiple_of(r, 8)
    pltpu.sync_copy(src_hbm.at[pl.ds(r, STAGE)], src_v)
    pltpu.sync_copy(dst_hbm.at[pl.ds(r, STAGE)], dst_v)
    pltpu.async_copy(table.at[src_v.at[0]], rows_a, s0)

    def pair(jj, carry):
        j0 = jj * 2
        j1 = j0 + 1
        pltpu.async_copy(table.at[src_v.at[j1]], rows_b, s1)
        pltpu.make_async_copy(table.at[src_v.at[j0]], rows_a, s0).wait()
        pltpu.sync_copy(rows_a, acc.at[dst_v.at[j0]], add=True)

        @pl.when(jj < (STAGE // 2) - 1)
        def _():
            pltpu.async_copy(table.at[src_v.at[j0 + 2]], rows_a, s0)

        pltpu.make_async_copy(table.at[src_v.at[j1]], rows_b, s1).wait()
        pltpu.sync_copy(rows_b, acc.at[dst_v.at[j1]], add=True)
        return carry
    lax.fori_loop(0, STAGE // 2, pair, 0)


def _l1_body(xa, xb, src_hbm, dst_hbm, z128,
             agg_a, agg_b,
             src_v, dst_v, rows_a, rows_b, acc, s0, s1):
    cid = lax.axis_index("c")
    sid = lax.axis_index("s")
    r0 = sid * CHUNKS_L1
    sl = pl.ds(sid * ROWS_PER_TILE, ROWS_PER_TILE)
    pltpu.sync_copy(z128, acc.at[sl])

    plsc.subcore_barrier()

    def _run(table):
        def stage_body(s, carry):
            _stream_stage(table, src_hbm, dst_hbm, r0 + s * STAGE,
                          acc, src_v, dst_v, rows_a, rows_b, s0, s1)
            return carry
        lax.fori_loop(0, CHUNKS_L1 // STAGE, stage_body, 0)

    @pl.when(cid == 0)
    def _():
        _run(xa)

    @pl.when(cid == 1)
    def _():
        _run(xb)

    plsc.subcore_barrier()

    @pl.when(cid == 0)
    def _():
        pltpu.sync_copy(acc.at[sl], agg_a.at[sl])

    @pl.when(cid == 1)
    def _():
        pltpu.sync_copy(acc.at[sl], agg_b.at[sl])


_l1_aggregate = pl.kernel(
    _l1_body,
    out_type=[
        jax.ShapeDtypeStruct((N_PAD, FH), jnp.float32),
        jax.ShapeDtypeStruct((N_PAD, FH), jnp.float32),
    ],
    mesh=_MESH,
    scratch_types=[
        pltpu.VMEM((STAGE, B_EDGE), jnp.int32),
        pltpu.VMEM((STAGE, B_EDGE), jnp.int32),
        pltpu.VMEM((B_EDGE, FH), jnp.float32),
        pltpu.VMEM((B_EDGE, FH), jnp.float32),
        pltpu.VMEM_SHARED((N_PAD, FH), jnp.float32),
        pltpu.SemaphoreType.DMA,
        pltpu.SemaphoreType.DMA,
    ],
)


def _l2_body(p_hbm, src_hbm, dst_hbm, z128,
             agg2_a, agg2_b,
             src_v, dst_v, rows_a, rows_b, acc, s0, s1):
    cid = lax.axis_index("c")
    sid = lax.axis_index("s")
    wid = sid * N_CORES + cid
    r0 = wid * CHUNKS_L2
    sl = pl.ds(sid * ROWS_PER_TILE, ROWS_PER_TILE)
    pltpu.sync_copy(z128, acc.at[sl])
    plsc.subcore_barrier()

    def stage_body(s, carry):
        _stream_stage(p_hbm, src_hbm, dst_hbm, r0 + s * STAGE,
                      acc, src_v, dst_v, rows_a, rows_b, s0, s1)
        return carry
    lax.fori_loop(0, CHUNKS_L2 // STAGE, stage_body, 0)

    plsc.subcore_barrier()

    @pl.when(cid == 0)
    def _():
        pltpu.sync_copy(acc.at[sl], agg2_a.at[sl])

    @pl.when(cid == 1)
    def _():
        pltpu.sync_copy(acc.at[sl], agg2_b.at[sl])


_l2_aggregate = pl.kernel(
    _l2_body,
    out_type=[
        jax.ShapeDtypeStruct((N_PAD, CP), jnp.float32),
        jax.ShapeDtypeStruct((N_PAD, CP), jnp.float32),
    ],
    mesh=_MESH,
    scratch_types=[
        pltpu.VMEM((STAGE, B_EDGE), jnp.int32),
        pltpu.VMEM((STAGE, B_EDGE), jnp.int32),
        pltpu.VMEM((B_EDGE, CP), jnp.float32),
        pltpu.VMEM((B_EDGE, CP), jnp.float32),
        pltpu.VMEM_SHARED((N_PAD, CP), jnp.float32),
        pltpu.SemaphoreType.DMA,
        pltpu.SemaphoreType.DMA,
    ],
)

ROW_BLK = 632
OUT_BLK = 2000


def _mlp_body(x_ref, agg_a_ref, agg_b_ref, ca_ref, cb_ref,
              w1l_ref, b1_ref, w1r_ref, w2l_ref, b2_ref, w2r_ref,
              p_ref, q_ref):
    inv = 1.0 / jnp.maximum(ca_ref[...] + cb_ref[...], 1.0)
    ha = agg_a_ref[...] * inv
    hb = agg_b_ref[...] * inv
    w1l = w1l_ref[...]
    s = jnp.dot(ha, w1l[0:FH, :], preferred_element_type=jnp.float32)
    s = s + jnp.dot(hb, w1l[FH:F_IN, :], preferred_element_type=jnp.float32)
    s = s + jnp.dot(x_ref[...], w1r_ref[...], preferred_element_type=jnp.float32)
    h = jnp.maximum(s + b1_ref[...], 0.0)
    p_ref[...] = jnp.dot(h, w2l_ref[...], preferred_element_type=jnp.float32)
    q_ref[...] = jnp.dot(h, w2r_ref[...], preferred_element_type=jnp.float32) + b2_ref[...]


def _out_body(a2a_ref, a2b_ref, ca_ref, cb_ref, q_ref, o_ref):
    inv = 1.0 / jnp.maximum(ca_ref[...] + cb_ref[...], 1.0)
    s = (a2a_ref[...] + a2b_ref[...]) * inv
    t = s[:, 0:C] + q_ref[...]
    m = jnp.max(t, axis=1, keepdims=True)
    lse = jnp.log(jnp.sum(jnp.exp(t - m), axis=1, keepdims=True)) + m
    o_ref[...] = t - lse


def kernel(x, edge_index, W1_l, b1_l, W1_r, W2_l, b2_l, W2_r):
    src = edge_index[0]
    dst = edge_index[1]
    pad = E_PAD - E
    ar = jnp.arange(pad, dtype=jnp.int32)
    src2 = jnp.concatenate([src, ar % N]).reshape(N_ROWS_IDX, B_EDGE)
    dst2 = jnp.concatenate([dst, N + (ar % (N_PAD - N))]).reshape(N_ROWS_IDX, B_EDGE)
    xa = x[:, :FH]
    xb = x[:, FH:]
    z128 = jnp.zeros((ROWS_PER_TILE, FH), jnp.float32)
    ones128 = jnp.ones((B_EDGE, 128), jnp.float32)

    cnt_a2, cnt_b2 = _cnt_aggregate(dst2, z128, ones128)
    ca = cnt_a2[:, 0:1]
    cb = cnt_b2[:, 0:1]
    agg_a, agg_b = _l1_aggregate(xa, xb, src2, dst2, z128)

    w2l_pad = jnp.concatenate([W2_l, jnp.zeros((H, CP - C), jnp.float32)], axis=1)
    b1r = b1_l.reshape(1, H)
    b2r = b2_l.reshape(1, C)

    p, q = pl.pallas_call(
        _mlp_body,
        grid=(N_PAD // ROW_BLK,),
        in_specs=[
            pl.BlockSpec((ROW_BLK, F_IN), lambda i: (i, 0)),
            pl.BlockSpec((ROW_BLK, FH), lambda i: (i, 0)),
            pl.BlockSpec((ROW_BLK, FH), lambda i: (i, 0)),
            pl.BlockSpec((ROW_BLK, 1), lambda i: (i, 0)),
            pl.BlockSpec((ROW_BLK, 1), lambda i: (i, 0)),
            pl.BlockSpec((F_IN, H), lambda i: (0, 0)),
            pl.BlockSpec((1, H), lambda i: (0, 0)),
            pl.BlockSpec((F_IN, H), lambda i: (0, 0)),
            pl.BlockSpec((H, CP), lambda i: (0, 0)),
            pl.BlockSpec((1, C), lambda i: (0, 0)),
            pl.BlockSpec((H, C), lambda i: (0, 0)),
        ],
        out_specs=[
            pl.BlockSpec((ROW_BLK, CP), lambda i: (i, 0)),
            pl.BlockSpec((ROW_BLK, C), lambda i: (i, 0)),
        ],
        out_shape=[
            jax.ShapeDtypeStruct((N_PAD, CP), jnp.float32),
            jax.ShapeDtypeStruct((N_PAD, C), jnp.float32),
        ],
    )(x, agg_a, agg_b, ca, cb, W1_l, b1r, W1_r, w2l_pad, b2r, W2_r)

    agg2_a, agg2_b = _l2_aggregate(p, src2, dst2, z128)

    out = pl.pallas_call(
        _out_body,
        grid=(N // OUT_BLK,),
        in_specs=[
            pl.BlockSpec((OUT_BLK, CP), lambda i: (i, 0)),
            pl.BlockSpec((OUT_BLK, CP), lambda i: (i, 0)),
            pl.BlockSpec((OUT_BLK, 1), lambda i: (i, 0)),
            pl.BlockSpec((OUT_BLK, 1), lambda i: (i, 0)),
            pl.BlockSpec((OUT_BLK, C), lambda i: (i, 0)),
        ],
        out_specs=pl.BlockSpec((OUT_BLK, C), lambda i: (i, 0)),
        out_shape=jax.ShapeDtypeStruct((N, C), jnp.float32),
    )(agg2_a, agg2_b, ca, cb, q)
    return out


# speedup vs baseline: 10.0820x; 1.1385x over previous
import jax
import jax.numpy as jnp
from jax import lax
from jax.experimental import pallas as pl
from jax.experimental.pallas import tpu as pltpu
from jax.experimental.pallas import tpu_sc as plsc

N = 10000
E = 160000
F_IN = 256
H = 256
C = 40

FH = 128
CP = 128
B_EDGE = 128
E_PAD = 163840
N_ROWS_IDX = E_PAD // B_EDGE
N_SUBCORES = 16
N_CORES = 2
CHUNKS_L1 = N_ROWS_IDX // N_SUBCORES
HALF_L1 = CHUNKS_L1 // 2
CHUNKS_L2 = N_ROWS_IDX // (N_SUBCORES * N_CORES)
STAGE = 40
N_PAD = 10112
ROWS_PER_TILE = N_PAD // N_SUBCORES

_MESH = plsc.VectorSubcoreMesh(core_axis_name="c", subcore_axis_name="s")


CNT_ROWS = 80


def _cnt_body(dst_hbm, zc,
              cnt_a, cnt_b,
              dst_v, lcnt_v, idx_v, cacc):
    cid = lax.axis_index("c")
    sid = lax.axis_index("s")
    wid = sid * N_CORES + cid
    r0 = wid * CHUNKS_L2
    pltpu.sync_copy(dst_hbm.at[pl.ds(r0, CHUNKS_L2)], dst_v)
    pltpu.sync_copy(zc, lcnt_v)
    for k in range(CNT_ROWS // 16):
        idx_v[0, pl.ds(16 * k, 16)] = lax.iota(jnp.int32, 16) + (16 * k)

    @pl.when(sid == 0)
    def _():
        pltpu.sync_copy(zc, cacc)

    plsc.subcore_barrier()
    ones = jnp.full((16,), 1.0, jnp.float32)

    def body(j, carry):
        for k in range(8):
            d = dst_v[j, pl.ds(16 * k, 16)]
            plsc.addupdate_scatter(
                lcnt_v, [lax.shift_right_logical(d, 7), d & 127], ones)
        return carry
    lax.fori_loop(0, CHUNKS_L2, body, 0)

    pltpu.sync_copy(lcnt_v, cacc.at[idx_v.at[0]], add=True)
    plsc.subcore_barrier()

    @pl.when((cid == 0) & (sid == 0))
    def _():
        pltpu.sync_copy(cacc, cnt_a)

    @pl.when((cid == 1) & (sid == 0))
    def _():
        pltpu.sync_copy(cacc, cnt_b)


_cnt_aggregate = pl.kernel(
    _cnt_body,
    out_type=[
        jax.ShapeDtypeStruct((CNT_ROWS, 128), jnp.float32),
        jax.ShapeDtypeStruct((CNT_ROWS, 128), jnp.float32),
    ],
    mesh=_MESH,
    compiler_params=pltpu.CompilerParams(needs_layout_passes=False),
    scratch_types=[
        pltpu.VMEM((CHUNKS_L2, B_EDGE), jnp.int32),
        pltpu.VMEM((CNT_ROWS, 128), jnp.float32),
        pltpu.VMEM((1, CNT_ROWS), jnp.int32),
        pltpu.VMEM_SHARED((CNT_ROWS, 128), jnp.float32),
    ],
)


def _stream_stage(table, src_hbm, dst_hbm, r, acc, src_v, dst_v, rows_a, rows_b, s0, s1):
    r = pl.multiple_of(r, 8)
    pltpu.sync_copy(src_hbm.at[pl.ds(r, STAGE)], src_v)
    pltpu.sync_copy(dst_hbm.at[pl.ds(r, STAGE)], dst_v)
    pltpu.async_copy(table.at[src_v.at[0]], rows_a, s0)

    def pair(jj, carry):
        j0 = jj * 2
        j1 = j0 + 1
        pltpu.async_copy(table.at[src_v.at[j1]], rows_b, s1)
        pltpu.make_async_copy(table.at[src_v.at[j0]], rows_a, s0).wait()
        pltpu.sync_copy(rows_a, acc.at[dst_v.at[j0]], add=True)

        @pl.when(jj < (STAGE // 2) - 1)
        def _():
            pltpu.async_copy(table.at[src_v.at[j0 + 2]], rows_a, s0)

        pltpu.make_async_copy(table.at[src_v.at[j1]], rows_b, s1).wait()
        pltpu.sync_copy(rows_b, acc.at[dst_v.at[j1]], add=True)
        return carry
    lax.fori_loop(0, STAGE // 2, pair, 0)


def _l1_body(xa, xb, src_hbm, dst_hbm, z128,
             agg_a, agg_b,
             src_v, dst_v, rows_a, rows_b, acc, s0, s1):
    cid = lax.axis_index("c")
    sid = lax.axis_index("s")
    r0 = sid * CHUNKS_L1
    sl = pl.ds(sid * ROWS_PER_TILE, ROWS_PER_TILE)
    pltpu.sync_copy(z128, acc.at[sl])

    plsc.subcore_barrier()

    def _run(table):
        def stage_body(s, carry):
            _stream_stage(table, src_hbm, dst_hbm, r0 + s * STAGE,
                          acc, src_v, dst_v, rows_a, rows_b, s0, s1)
            return carry
        lax.fori_loop(0, CHUNKS_L1 // STAGE, stage_body, 0)

    @pl.when(cid == 0)
    def _():
        _run(xa)

    @pl.when(cid == 1)
    def _():
        _run(xb)

    plsc.subcore_barrier()

    @pl.when(cid == 0)
    def _():
        pltpu.sync_copy(acc.at[sl], agg_a.at[sl])

    @pl.when(cid == 1)
    def _():
        pltpu.sync_copy(acc.at[sl], agg_b.at[sl])


_l1_aggregate = pl.kernel(
    _l1_body,
    out_type=[
        jax.ShapeDtypeStruct((N_PAD, FH), jnp.float32),
        jax.ShapeDtypeStruct((N_PAD, FH), jnp.float32),
    ],
    mesh=_MESH,
    scratch_types=[
        pltpu.VMEM((STAGE, B_EDGE), jnp.int32),
        pltpu.VMEM((STAGE, B_EDGE), jnp.int32),
        pltpu.VMEM((B_EDGE, FH), jnp.float32),
        pltpu.VMEM((B_EDGE, FH), jnp.float32),
        pltpu.VMEM_SHARED((N_PAD, FH), jnp.float32),
        pltpu.SemaphoreType.DMA,
        pltpu.SemaphoreType.DMA,
    ],
)


def _l2_body(p_hbm, src_hbm, dst_hbm, z128,
             agg2_a, agg2_b,
             src_v, dst_v, rows_a, rows_b, acc, s0, s1):
    cid = lax.axis_index("c")
    sid = lax.axis_index("s")
    wid = sid * N_CORES + cid
    r0 = wid * CHUNKS_L2
    sl = pl.ds(sid * ROWS_PER_TILE, ROWS_PER_TILE)
    pltpu.sync_copy(z128, acc.at[sl])
    plsc.subcore_barrier()

    def stage_body(s, carry):
        _stream_stage(p_hbm, src_hbm, dst_hbm, r0 + s * STAGE,
                      acc, src_v, dst_v, rows_a, rows_b, s0, s1)
        return carry
    lax.fori_loop(0, CHUNKS_L2 // STAGE, stage_body, 0)

    plsc.subcore_barrier()

    @pl.when(cid == 0)
    def _():
        pltpu.sync_copy(acc.at[sl], agg2_a.at[sl])

    @pl.when(cid == 1)
    def _():
        pltpu.sync_copy(acc.at[sl], agg2_b.at[sl])


_l2_aggregate = pl.kernel(
    _l2_body,
    out_type=[
        jax.ShapeDtypeStruct((N_PAD, CP), jnp.float32),
        jax.ShapeDtypeStruct((N_PAD, CP), jnp.float32),
    ],
    mesh=_MESH,
    scratch_types=[
        pltpu.VMEM((STAGE, B_EDGE), jnp.int32),
        pltpu.VMEM((STAGE, B_EDGE), jnp.int32),
        pltpu.VMEM((B_EDGE, CP), jnp.float32),
        pltpu.VMEM((B_EDGE, CP), jnp.float32),
        pltpu.VMEM_SHARED((N_PAD, CP), jnp.float32),
        pltpu.SemaphoreType.DMA,
        pltpu.SemaphoreType.DMA,
    ],
)

ROW_BLK = 632
OUT_BLK = 2000


def _mlp_body(x_ref, agg_a_ref, agg_b_ref, ca_ref, cb_ref,
              w1l_ref, b1_ref, w1r_ref, w2l_ref, b2_ref, w2r_ref,
              p_ref, q_ref):
    inv = 1.0 / jnp.maximum(ca_ref[...] + cb_ref[...], 1.0)
    ha = agg_a_ref[...] * inv
    hb = agg_b_ref[...] * inv
    w1l = w1l_ref[...]
    s = jnp.dot(ha, w1l[0:FH, :], preferred_element_type=jnp.float32)
    s = s + jnp.dot(hb, w1l[FH:F_IN, :], preferred_element_type=jnp.float32)
    s = s + jnp.dot(x_ref[...], w1r_ref[...], preferred_element_type=jnp.float32)
    h = jnp.maximum(s + b1_ref[...], 0.0)
    p_ref[...] = jnp.dot(h, w2l_ref[...], preferred_element_type=jnp.float32)
    q_ref[...] = jnp.dot(h, w2r_ref[...], preferred_element_type=jnp.float32) + b2_ref[...]


def _out_body(a2a_ref, a2b_ref, ca_ref, cb_ref, q_ref, o_ref):
    inv = 1.0 / jnp.maximum(ca_ref[...] + cb_ref[...], 1.0)
    s = (a2a_ref[...] + a2b_ref[...]) * inv
    t = s[:, 0:C] + q_ref[...]
    m = jnp.max(t, axis=1, keepdims=True)
    lse = jnp.log(jnp.sum(jnp.exp(t - m), axis=1, keepdims=True)) + m
    o_ref[...] = t - lse


def kernel(x, edge_index, W1_l, b1_l, W1_r, W2_l, b2_l, W2_r):
    src = edge_index[0]
    dst = edge_index[1]
    pad = E_PAD - E
    ar = jnp.arange(pad, dtype=jnp.int32)
    src2 = jnp.concatenate([src, ar % N]).reshape(N_ROWS_IDX, B_EDGE)
    dst2 = jnp.concatenate([dst, N + (ar % (N_PAD - N))]).reshape(N_ROWS_IDX, B_EDGE)
    xa = x[:, :FH]
    xb = x[:, FH:]
    z128 = jnp.zeros((ROWS_PER_TILE, FH), jnp.float32)
    zc = jnp.zeros((CNT_ROWS, 128), jnp.float32)

    cnt_a2, cnt_b2 = _cnt_aggregate(dst2, zc)
    ca = cnt_a2.reshape(CNT_ROWS * 128, 1)[:N_PAD]
    cb = cnt_b2.reshape(CNT_ROWS * 128, 1)[:N_PAD]
    agg_a, agg_b = _l1_aggregate(xa, xb, src2, dst2, z128)

    w2l_pad = jnp.concatenate([W2_l, jnp.zeros((H, CP - C), jnp.float32)], axis=1)
    b1r = b1_l.reshape(1, H)
    b2r = b2_l.reshape(1, C)

    p, q = pl.pallas_call(
        _mlp_body,
        grid=(N_PAD // ROW_BLK,),
        in_specs=[
            pl.BlockSpec((ROW_BLK, F_IN), lambda i: (i, 0)),
            pl.BlockSpec((ROW_BLK, FH), lambda i: (i, 0)),
            pl.BlockSpec((ROW_BLK, FH), lambda i: (i, 0)),
            pl.BlockSpec((ROW_BLK, 1), lambda i: (i, 0)),
            pl.BlockSpec((ROW_BLK, 1), lambda i: (i, 0)),
            pl.BlockSpec((F_IN, H), lambda i: (0, 0)),
            pl.BlockSpec((1, H), lambda i: (0, 0)),
            pl.BlockSpec((F_IN, H), lambda i: (0, 0)),
            pl.BlockSpec((H, CP), lambda i: (0, 0)),
            pl.BlockSpec((1, C), lambda i: (0, 0)),
            pl.BlockSpec((H, C), lambda i: (0, 0)),
        ],
        out_specs=[
            pl.BlockSpec((ROW_BLK, CP), lambda i: (i, 0)),
            pl.BlockSpec((ROW_BLK, C), lambda i: (i, 0)),
        ],
        out_shape=[
            jax.ShapeDtypeStruct((N_PAD, CP), jnp.float32),
            jax.ShapeDtypeStruct((N_PAD, C), jnp.float32),
        ],
    )(x, agg_a, agg_b, ca, cb, W1_l, b1r, W1_r, w2l_pad, b2r, W2_r)

    agg2_a, agg2_b = _l2_aggregate(p, src2, dst2, z128)

    out = pl.pallas_call(
        _out_body,
        grid=(N // OUT_BLK,),
        in_specs=[
            pl.BlockSpec((OUT_BLK, CP), lambda i: (i, 0)),
            pl.BlockSpec((OUT_BLK, CP), lambda i: (i, 0)),
            pl.BlockSpec((OUT_BLK, 1), lambda i: (i, 0)),
            pl.BlockSpec((OUT_BLK, 1), lambda i: (i, 0)),
            pl.BlockSpec((OUT_BLK, C), lambda i: (i, 0)),
        ],
        out_specs=pl.BlockSpec((OUT_BLK, C), lambda i: (i, 0)),
        out_shape=jax.ShapeDtypeStruct((N, C), jnp.float32),
    )(agg2_a, agg2_b, ca, cb, q)
    return out


# speedup vs baseline: 10.4053x; 1.0321x over previous
import jax
import jax.numpy as jnp
from jax import lax
from jax.experimental import pallas as pl
from jax.experimental.pallas import tpu as pltpu
from jax.experimental.pallas import tpu_sc as plsc

N = 10000
E = 160000
F_IN = 256
H = 256
C = 40

FH = 128
CP = 128
B_EDGE = 128
E_PAD = 163840
N_ROWS_IDX = E_PAD // B_EDGE
N_SUBCORES = 16
N_CORES = 2
CHUNKS_L1 = N_ROWS_IDX // N_SUBCORES
HALF_L1 = CHUNKS_L1 // 2
CHUNKS_L2 = N_ROWS_IDX // (N_SUBCORES * N_CORES)
STAGE = 40
N_PAD = 10112
ROWS_PER_TILE = N_PAD // N_SUBCORES

_MESH = plsc.VectorSubcoreMesh(core_axis_name="c", subcore_axis_name="s")


CNT_ROWS = 80


def _cnt_body(dst_hbm, zc,
              cnt_a, cnt_b,
              dst_v, lcnt_v, idx_v, cacc):
    cid = lax.axis_index("c")
    sid = lax.axis_index("s")
    wid = sid * N_CORES + cid
    r0 = wid * CHUNKS_L2
    pltpu.sync_copy(dst_hbm.at[pl.ds(r0, CHUNKS_L2)], dst_v)
    pltpu.sync_copy(zc, lcnt_v)
    for k in range(CNT_ROWS // 16):
        idx_v[0, pl.ds(16 * k, 16)] = lax.iota(jnp.int32, 16) + (16 * k)

    @pl.when(sid == 0)
    def _():
        pltpu.sync_copy(zc, cacc)

    plsc.subcore_barrier()
    ones = jnp.full((16,), 1.0, jnp.float32)

    def body(j, carry):
        for k in range(8):
            d = dst_v[j, pl.ds(16 * k, 16)]
            plsc.addupdate_scatter(
                lcnt_v, [lax.shift_right_logical(d, 7), d & 127], ones)
        return carry
    lax.fori_loop(0, CHUNKS_L2, body, 0)

    pltpu.sync_copy(lcnt_v, cacc.at[idx_v.at[0]], add=True)
    plsc.subcore_barrier()

    @pl.when((cid == 0) & (sid == 0))
    def _():
        pltpu.sync_copy(cacc, cnt_a)

    @pl.when((cid == 1) & (sid == 0))
    def _():
        pltpu.sync_copy(cacc, cnt_b)


_cnt_aggregate = pl.kernel(
    _cnt_body,
    out_type=[
        jax.ShapeDtypeStruct((CNT_ROWS, 128), jnp.float32),
        jax.ShapeDtypeStruct((CNT_ROWS, 128), jnp.float32),
    ],
    mesh=_MESH,
    compiler_params=pltpu.CompilerParams(needs_layout_passes=False),
    scratch_types=[
        pltpu.VMEM((CHUNKS_L2, B_EDGE), jnp.int32),
        pltpu.VMEM((CNT_ROWS, 128), jnp.float32),
        pltpu.VMEM((1, CNT_ROWS), jnp.int32),
        pltpu.VMEM_SHARED((CNT_ROWS, 128), jnp.float32),
    ],
)


def _stream_stage(table, src_hbm, dst_hbm, r, acc, src_v, dst_v, rows_a, rows_b, s0, s1):
    r = pl.multiple_of(r, 8)
    pltpu.sync_copy(src_hbm.at[pl.ds(r, STAGE)], src_v)
    pltpu.sync_copy(dst_hbm.at[pl.ds(r, STAGE)], dst_v)
    pltpu.async_copy(table.at[src_v.at[0]], rows_a, s0)

    def pair(jj, carry):
        j0 = jj * 2
        j1 = j0 + 1
        pltpu.async_copy(table.at[src_v.at[j1]], rows_b, s1)
        pltpu.make_async_copy(table.at[src_v.at[j0]], rows_a, s0).wait()
        pltpu.sync_copy(rows_a, acc.at[dst_v.at[j0]], add=True)

        @pl.when(jj < (STAGE // 2) - 1)
        def _():
            pltpu.async_copy(table.at[src_v.at[j0 + 2]], rows_a, s0)

        pltpu.make_async_copy(table.at[src_v.at[j1]], rows_b, s1).wait()
        pltpu.sync_copy(rows_b, acc.at[dst_v.at[j1]], add=True)
        return carry
    lax.fori_loop(0, STAGE // 2, pair, 0)


def _l1_body(xa, xb, src_hbm, dst_hbm, z128,
             agg_a, agg_b,
             src_v, dst_v, rows_a, rows_b, acc, s0, s1):
    cid = lax.axis_index("c")
    sid = lax.axis_index("s")
    r0 = sid * CHUNKS_L1
    sl = pl.ds(sid * ROWS_PER_TILE, ROWS_PER_TILE)
    pltpu.sync_copy(z128, acc.at[sl])

    plsc.subcore_barrier()

    def _run(table):
        def stage_body(s, carry):
            _stream_stage(table, src_hbm, dst_hbm, r0 + s * STAGE,
                          acc, src_v, dst_v, rows_a, rows_b, s0, s1)
            return carry
        lax.fori_loop(0, CHUNKS_L1 // STAGE, stage_body, 0)

    @pl.when(cid == 0)
    def _():
        _run(xa)

    @pl.when(cid == 1)
    def _():
        _run(xb)

    plsc.subcore_barrier()

    @pl.when(cid == 0)
    def _():
        pltpu.sync_copy(acc.at[sl], agg_a.at[sl])

    @pl.when(cid == 1)
    def _():
        pltpu.sync_copy(acc.at[sl], agg_b.at[sl])


_l1_aggregate = pl.kernel(
    _l1_body,
    out_type=[
        jax.ShapeDtypeStruct((N_PAD, FH), jnp.float32),
        jax.ShapeDtypeStruct((N_PAD, FH), jnp.float32),
    ],
    mesh=_MESH,
    scratch_types=[
        pltpu.VMEM((STAGE, B_EDGE), jnp.int32),
        pltpu.VMEM((STAGE, B_EDGE), jnp.int32),
        pltpu.VMEM((B_EDGE, FH), jnp.float32),
        pltpu.VMEM((B_EDGE, FH), jnp.float32),
        pltpu.VMEM_SHARED((N_PAD, FH), jnp.float32),
        pltpu.SemaphoreType.DMA,
        pltpu.SemaphoreType.DMA,
    ],
)


def _l2_body(p_hbm, src_hbm, dst_hbm, z128,
             agg2_a, agg2_b,
             src_v, dst_v, rows_a, rows_b, acc, s0, s1):
    cid = lax.axis_index("c")
    sid = lax.axis_index("s")
    wid = sid * N_CORES + cid
    r0 = wid * CHUNKS_L2
    sl = pl.ds(sid * ROWS_PER_TILE, ROWS_PER_TILE)
    pltpu.sync_copy(z128, acc.at[sl])
    plsc.subcore_barrier()

    def stage_body(s, carry):
        _stream_stage(p_hbm, src_hbm, dst_hbm, r0 + s * STAGE,
                      acc, src_v, dst_v, rows_a, rows_b, s0, s1)
        return carry
    lax.fori_loop(0, CHUNKS_L2 // STAGE, stage_body, 0)

    plsc.subcore_barrier()

    @pl.when(cid == 0)
    def _():
        pltpu.sync_copy(acc.at[sl], agg2_a.at[sl])

    @pl.when(cid == 1)
    def _():
        pltpu.sync_copy(acc.at[sl], agg2_b.at[sl])


_l2_aggregate = pl.kernel(
    _l2_body,
    out_type=[
        jax.ShapeDtypeStruct((N_PAD, CP), jnp.float32),
        jax.ShapeDtypeStruct((N_PAD, CP), jnp.float32),
    ],
    mesh=_MESH,
    scratch_types=[
        pltpu.VMEM((STAGE, B_EDGE), jnp.int32),
        pltpu.VMEM((STAGE, B_EDGE), jnp.int32),
        pltpu.VMEM((B_EDGE, CP), jnp.float32),
        pltpu.VMEM((B_EDGE, CP), jnp.float32),
        pltpu.VMEM_SHARED((N_PAD, CP), jnp.float32),
        pltpu.SemaphoreType.DMA,
        pltpu.SemaphoreType.DMA,
    ],
)

ROW_BLK = 1264
OUT_BLK = 2000


def _mlp_body(x_ref, agg_a_ref, agg_b_ref, ca_ref, cb_ref,
              w1l_ref, b1_ref, w1r_ref, w2_ref, b2_ref,
              p_ref):
    inv = 1.0 / jnp.maximum(ca_ref[...] + cb_ref[...], 1.0)
    ha = agg_a_ref[...] * inv
    hb = agg_b_ref[...] * inv
    w1l = w1l_ref[...]
    s = jnp.dot(ha, w1l[0:FH, :], preferred_element_type=jnp.float32)
    s = s + jnp.dot(hb, w1l[FH:F_IN, :], preferred_element_type=jnp.float32)
    s = s + jnp.dot(x_ref[...], w1r_ref[...], preferred_element_type=jnp.float32)
    h = jnp.maximum(s + b1_ref[...], 0.0)
    p_ref[...] = jnp.dot(h, w2_ref[...], preferred_element_type=jnp.float32) + b2_ref[...]


def _out_body(a2a_ref, a2b_ref, ca_ref, cb_ref, p_ref, o_ref):
    inv = 1.0 / jnp.maximum(ca_ref[...] + cb_ref[...], 1.0)
    s = (a2a_ref[...] + a2b_ref[...]) * inv
    t = s[:, 0:C] + p_ref[:, C:2 * C]
    m = jnp.max(t, axis=1, keepdims=True)
    lse = jnp.log(jnp.sum(jnp.exp(t - m), axis=1, keepdims=True)) + m
    o_ref[...] = t - lse


def kernel(x, edge_index, W1_l, b1_l, W1_r, W2_l, b2_l, W2_r):
    src = edge_index[0]
    dst = edge_index[1]
    pad = E_PAD - E
    ar = jnp.arange(pad, dtype=jnp.int32)
    src2 = jnp.concatenate([src, ar % N]).reshape(N_ROWS_IDX, B_EDGE)
    dst2 = jnp.concatenate([dst, N + (ar % (N_PAD - N))]).reshape(N_ROWS_IDX, B_EDGE)
    xa = x[:, :FH]
    xb = x[:, FH:]
    z128 = jnp.zeros((ROWS_PER_TILE, FH), jnp.float32)
    zc = jnp.zeros((CNT_ROWS, 128), jnp.float32)

    cnt_a2, cnt_b2 = _cnt_aggregate(dst2, zc)
    ca = cnt_a2.reshape(CNT_ROWS * 128, 1)[:N_PAD]
    cb = cnt_b2.reshape(CNT_ROWS * 128, 1)[:N_PAD]
    agg_a, agg_b = _l1_aggregate(xa, xb, src2, dst2, z128)

    w2_pad = jnp.concatenate(
        [W2_l, W2_r, jnp.zeros((H, CP - 2 * C), jnp.float32)], axis=1)
    b1r = b1_l.reshape(1, H)
    b2r = jnp.concatenate(
        [jnp.zeros((C,), jnp.float32), b2_l,
         jnp.zeros((CP - 2 * C,), jnp.float32)]).reshape(1, CP)

    p = pl.pallas_call(
        _mlp_body,
        grid=(N_PAD // ROW_BLK,),
        in_specs=[
            pl.BlockSpec((ROW_BLK, F_IN), lambda i: (i, 0)),
            pl.BlockSpec((ROW_BLK, FH), lambda i: (i, 0)),
            pl.BlockSpec((ROW_BLK, FH), lambda i: (i, 0)),
            pl.BlockSpec((ROW_BLK, 1), lambda i: (i, 0)),
            pl.BlockSpec((ROW_BLK, 1), lambda i: (i, 0)),
            pl.BlockSpec((F_IN, H), lambda i: (0, 0)),
            pl.BlockSpec((1, H), lambda i: (0, 0)),
            pl.BlockSpec((F_IN, H), lambda i: (0, 0)),
            pl.BlockSpec((H, CP), lambda i: (0, 0)),
            pl.BlockSpec((1, CP), lambda i: (0, 0)),
        ],
        out_specs=pl.BlockSpec((ROW_BLK, CP), lambda i: (i, 0)),
        out_shape=jax.ShapeDtypeStruct((N_PAD, CP), jnp.float32),
    )(x, agg_a, agg_b, ca, cb, W1_l, b1r, W1_r, w2_pad, b2r)

    agg2_a, agg2_b = _l2_aggregate(p, src2, dst2, z128)

    out = pl.pallas_call(
        _out_body,
        grid=(N // OUT_BLK,),
        in_specs=[
            pl.BlockSpec((OUT_BLK, CP), lambda i: (i, 0)),
            pl.BlockSpec((OUT_BLK, CP), lambda i: (i, 0)),
            pl.BlockSpec((OUT_BLK, 1), lambda i: (i, 0)),
            pl.BlockSpec((OUT_BLK, 1), lambda i: (i, 0)),
            pl.BlockSpec((OUT_BLK, CP), lambda i: (i, 0)),
        ],
        out_specs=pl.BlockSpec((OUT_BLK, C), lambda i: (i, 0)),
        out_shape=jax.ShapeDtypeStruct((N, C), jnp.float32),
    )(agg2_a, agg2_b, ca, cb, p)
    return out
